# baseline (speedup 1.0000x reference)
import functools

import jax
import jax.numpy as jnp
from jax import lax
from jax.experimental import pallas as pl
from jax.experimental.pallas import tpu as pltpu
from jax.experimental.pallas import tpu_sc as plsc

N = 10000
E = 160000
NC, NS = 2, 16
EG = 64
NG0, NG1 = 48, 112
NG_MAX = max(NG0, NG1)
NG_MIN = min(NG0, NG1)
NG_EXT = NG_MAX - NG_MIN
BIG_CORE = 0 if NG0 >= NG1 else 1
IDXR = NG_MAX
IDXB = max(NG_MIN, NG_EXT)
NBUF = 3
ROWS_PER_TILE = 640
NACC = NS * ROWS_PER_TILE

_mesh = plsc.VectorSubcoreMesh(core_axis_name="c", subcore_axis_name="s")


def _deg_body(dst3_hbm, znode_hbm, ones_hbm, out_hbm, idx_v, ones_v, buf_v,
              acc_s):
    c = lax.axis_index("c")
    s = lax.axis_index("s")
    ng = jnp.where(c == 0, NG0, NG1)
    r0 = s * ROWS_PER_TILE
    pltpu.sync_copy(znode_hbm.at[pl.ds(r0, ROWS_PER_TILE)], buf_v)
    pltpu.sync_copy(buf_v, acc_s.at[pl.ds(r0, ROWS_PER_TILE)])
    pltpu.sync_copy(ones_hbm, ones_v)
    pltpu.sync_copy(dst3_hbm.at[c, s], idx_v)
    plsc.subcore_barrier()

    def step(j, carry):
        pltpu.sync_copy(ones_v, acc_s.at[idx_v.at[j]], add=True)
        return carry

    lax.fori_loop(0, ng, step, 0)
    plsc.subcore_barrier()
    pltpu.sync_copy(acc_s.at[pl.ds(r0, ROWS_PER_TILE)], buf_v)
    pltpu.sync_copy(buf_v, out_hbm.at[c, pl.ds(r0, ROWS_PER_TILE)])


_sc_deg = pl.kernel(
    _deg_body,
    out_type=jax.ShapeDtypeStruct((NC, NACC), jnp.float32),
    mesh=_mesh,
    scratch_types=[
        pltpu.VMEM((IDXR, EG), jnp.int32),
        pltpu.VMEM((EG,), jnp.float32),
        pltpu.VMEM((ROWS_PER_TILE,), jnp.float32),
        pltpu.VMEM_SHARED((NACC,), jnp.float32),
    ],
    name="sc_degree",
)


def _spmm_body(h_hbm, src3_hbm, dst3_hbm, zrow_hbm, out_hbm,
               srcv, dstv, rows0, rows1, rows2, g0, g1, g2, s0, s1, s2,
               acc_s):
    c = lax.axis_index("c")
    s = lax.axis_index("s")
    r0 = s * ROWS_PER_TILE
    pltpu.sync_copy(zrow_hbm, rows0)
    for k in range(ROWS_PER_TILE // EG):
        pltpu.sync_copy(rows0, acc_s.at[pl.ds(r0 + k * EG, EG)])
    pltpu.sync_copy(src3_hbm.at[c, s, pl.ds(0, NG_MIN)],
                    srcv.at[pl.ds(0, NG_MIN)])
    pltpu.sync_copy(dst3_hbm.at[c, s, pl.ds(0, NG_MIN)],
                    dstv.at[pl.ds(0, NG_MIN)])
    plsc.subcore_barrier()
    rows = (rows0, rows1, rows2)
    gsems = (g0, g1, g2)
    ssems = (s0, s1, s2)

    def pipeline(n):
        gcp = [None] * NBUF
        scp = [None] * NBUF
        for t in range(min(2, n)):
            gcp[t % NBUF] = pltpu.async_copy(
                h_hbm.at[srcv.at[t]], rows[t % NBUF], gsems[t % NBUF])
        for j in range(n):
            b = j % NBUF
            gcp[b].wait()
            scp[b] = pltpu.async_copy(rows[b], acc_s.at[dstv.at[j]],
                                      ssems[b], add=True)
            nj = j + 2
            if nj < n:
                b2 = nj % NBUF
                if scp[b2] is not None:
                    scp[b2].wait()
                gcp[b2] = pltpu.async_copy(h_hbm.at[srcv.at[nj]], rows[b2],
                                           gsems[b2])
        for w in range(max(n - 3, 0), n):
            scp[w % NBUF].wait()

    pipeline(NG_MIN)

    if NG_EXT:
        @pl.when(c == BIG_CORE)
        def _():
            pltpu.sync_copy(src3_hbm.at[c, s, pl.ds(NG_MIN, NG_EXT)],
                            srcv.at[pl.ds(0, NG_EXT)])
            pltpu.sync_copy(dst3_hbm.at[c, s, pl.ds(NG_MIN, NG_EXT)],
                            dstv.at[pl.ds(0, NG_EXT)])
            pipeline(NG_EXT)

    plsc.subcore_barrier()
    for k in range(ROWS_PER_TILE // EG):
        pltpu.sync_copy(acc_s.at[pl.ds(r0 + k * EG, EG)], rows0)
        pltpu.sync_copy(rows0, out_hbm.at[c, pl.ds(r0 + k * EG, EG)])


def _make_spmm(d):
    params = None
    if d != 128:
        params = pltpu.CompilerParams(use_tc_tiling_on_sc=False)
    return pl.kernel(
        _spmm_body,
        out_type=jax.ShapeDtypeStruct((NC, NACC, d), jnp.float32),
        mesh=_mesh,
        scratch_types=(
            [pltpu.VMEM((IDXB, EG), jnp.int32)] * 2
            + [pltpu.VMEM((EG, d), jnp.float32)] * NBUF
            + [pltpu.SemaphoreType.DMA] * (2 * NBUF)
            + [pltpu.VMEM_SHARED((NACC, d), jnp.float32)]
        ),
        compiler_params=params,
        name=f"sc_spmm_{d}",
    )


_sc_spmm = {64: _make_spmm(64), 128: _make_spmm(128)}


def _dis_body(deg_ref, o_ref):
    d = jnp.sum(deg_ref[...], axis=0) + 1.0
    o_ref[...] = lax.rsqrt(d)


def _tc_dis(deg2):
    nr = NACC // 128
    return pl.pallas_call(
        _dis_body,
        out_shape=jax.ShapeDtypeStruct((nr, 128), jnp.float32),
    )(deg2.reshape(NC, nr, 128))


def _mm_body(x_ref, w_ref, dis_ref, o_ref):
    p = lax.dot_general(x_ref[...], w_ref[...], (((1,), (0,)), ((), ())),
                        preferred_element_type=jnp.float32,
                        precision=lax.Precision.HIGHEST)
    o_ref[...] = p * dis_ref[...]


def _tc_mm(x, w, dis, bm=1000):
    m, din = x.shape
    dout = w.shape[1]
    return pl.pallas_call(
        _mm_body,
        grid=(m // bm,),
        in_specs=[
            pl.BlockSpec((bm, din), lambda i: (i, 0)),
            pl.BlockSpec((din, dout), lambda i: (0, 0)),
            pl.BlockSpec((bm, 1), lambda i: (i, 0)),
        ],
        out_specs=pl.BlockSpec((bm, dout), lambda i: (i, 0)),
        out_shape=jax.ShapeDtypeStruct((m, dout), jnp.float32),
    )(x, w, dis)


def _comb_body(s0_ref, s1_ref, t_ref, dis_ref, b_ref, o_ref, *, act):
    v = (s0_ref[0] + s1_ref[0] + t_ref[...]) * dis_ref[...] + b_ref[...]
    if act:
        v = jnp.maximum(v, 0.0)
    o_ref[...] = v


def _tc_comb(sp, t, dis, b, act, bm=1000):
    m, d = t.shape
    return pl.pallas_call(
        functools.partial(_comb_body, act=act),
        grid=(m // bm,),
        in_specs=[
            pl.BlockSpec((1, bm, d), lambda i: (0, i, 0)),
            pl.BlockSpec((1, bm, d), lambda i: (1, i, 0)),
            pl.BlockSpec((bm, d), lambda i: (i, 0)),
            pl.BlockSpec((bm, 1), lambda i: (i, 0)),
            pl.BlockSpec((1, d), lambda i: (0, 0)),
        ],
        out_specs=pl.BlockSpec((bm, d), lambda i: (i, 0)),
        out_shape=jax.ShapeDtypeStruct((m, d), jnp.float32),
    )(sp, sp, t, dis, b)


def _adj_body(ai_ref, aj_ref, o_ref):
    p = lax.dot_general(ai_ref[...], aj_ref[...], (((1,), (1,)), ((), ())),
                        preferred_element_type=jnp.float32,
                        precision=lax.Precision.HIGHEST)
    o_ref[...] = jax.nn.sigmoid(p)


def _tc_adj(a, bm=400):
    m, d = a.shape
    return pl.pallas_call(
        _adj_body,
        grid=(m // bm,),
        in_specs=[
            pl.BlockSpec((bm, d), lambda i: (i, 0)),
            pl.BlockSpec((m, d), lambda i: (0, 0)),
        ],
        out_specs=pl.BlockSpec((bm, m), lambda i: (i, 0)),
        out_shape=jax.ShapeDtypeStruct((m, m), jnp.float32),
    )(a, a)


def kernel(x, edge_index, W1, b1, W2, b2, W3, b3, W4, b4, W5, b5):
    ei = edge_index.astype(jnp.int32)
    cap0 = NS * NG0 * EG
    cap1 = NS * NG1 * EG
    padl = cap0 + cap1 - E

    def split3(v, fill):
        vp = jnp.concatenate([v, fill])
        p0 = vp[:cap0].reshape(NS, NG0, EG)
        p1 = vp[cap0:].reshape(NS, NG1, EG)
        p0 = jnp.pad(p0, ((0, 0), (0, IDXR - NG0), (0, 0)))
        p1 = jnp.pad(p1, ((0, 0), (0, IDXR - NG1), (0, 0)))
        return jnp.stack([p0, p1])

    src3 = split3(ei[0], jnp.zeros((padl,), jnp.int32))
    dst3 = split3(ei[1], N + jnp.arange(padl, dtype=jnp.int32) % (NACC - N))
    znode = jnp.zeros((NACC,), jnp.float32)
    ones_eg = jnp.ones((EG,), jnp.float32)
    zrow = {64: jnp.zeros((EG, 64), jnp.float32),
            128: jnp.zeros((EG, 128), jnp.float32)}

    deg2 = _sc_deg(dst3, znode, ones_eg)
    dis = _tc_dis(deg2).reshape(NACC)[:N].reshape(N, 1)

    def conv(h, W, b, act):
        d = W.shape[1]
        t = _tc_mm(h, W, dis)
        sp = _sc_spmm[d](t, src3, dst3, zrow[d])
        return _tc_comb(sp, t, dis, b.reshape(1, d), act)

    z = conv(x, W1, b1, True)
    z = conv(z, W2, b2, True)
    h3 = conv(z, W3, b3, True)
    x_hat = conv(h3, W4, b4, False)
    a = conv(z, W5, b5, True)
    adj_hat = _tc_adj(a)
    return (x_hat, adj_hat)


# speedup vs baseline: 1.3204x; 1.3204x over previous
import functools

import jax
import jax.numpy as jnp
from jax import lax
from jax.experimental import pallas as pl
from jax.experimental.pallas import tpu as pltpu
from jax.experimental.pallas import tpu_sc as plsc

N = 10000
E = 160000
NC, NS = 2, 16
EG = 64
NG0, NG1 = 79, 79
NG_MAX = max(NG0, NG1)
NG_MIN = min(NG0, NG1)
NG_EXT = NG_MAX - NG_MIN
BIG_CORE = 0 if NG0 >= NG1 else 1
IDXR = NG_MAX
IDXB = max(NG_MIN, NG_EXT)
NBUF = 3
ROWS_PER_TILE = 640
NACC = NS * ROWS_PER_TILE

_mesh = plsc.VectorSubcoreMesh(core_axis_name="c", subcore_axis_name="s")


def _deg_body(dst3_hbm, znode_hbm, ones_hbm, out_hbm, idx_v, ones_v, buf_v,
              acc_s):
    c = lax.axis_index("c")
    s = lax.axis_index("s")
    ng = jnp.where(c == 0, NG0, NG1)
    r0 = s * ROWS_PER_TILE
    pltpu.sync_copy(znode_hbm.at[pl.ds(r0, ROWS_PER_TILE)], buf_v)
    pltpu.sync_copy(buf_v, acc_s.at[pl.ds(r0, ROWS_PER_TILE)])
    pltpu.sync_copy(ones_hbm, ones_v)
    pltpu.sync_copy(dst3_hbm.at[c, s], idx_v)
    plsc.subcore_barrier()

    def step(j, carry):
        pltpu.sync_copy(ones_v, acc_s.at[idx_v.at[j]], add=True)
        return carry

    lax.fori_loop(0, ng, step, 0)
    plsc.subcore_barrier()
    pltpu.sync_copy(acc_s.at[pl.ds(r0, ROWS_PER_TILE)], buf_v)
    pltpu.sync_copy(buf_v, out_hbm.at[c, pl.ds(r0, ROWS_PER_TILE)])


_sc_deg = pl.kernel(
    _deg_body,
    out_type=jax.ShapeDtypeStruct((NC, NACC), jnp.float32),
    mesh=_mesh,
    scratch_types=[
        pltpu.VMEM((IDXR, EG), jnp.int32),
        pltpu.VMEM((EG,), jnp.float32),
        pltpu.VMEM((ROWS_PER_TILE,), jnp.float32),
        pltpu.VMEM_SHARED((NACC,), jnp.float32),
    ],
    name="sc_degree",
)


def _spmm_body(h_hbm, src3_hbm, dst3_hbm, zrow_hbm, out_hbm,
               srcv, dstv, rows0, rows1, rows2, g0, g1, g2, s0, s1, s2,
               acc_s):
    c = lax.axis_index("c")
    s = lax.axis_index("s")
    r0 = s * ROWS_PER_TILE
    pltpu.sync_copy(zrow_hbm, rows0)
    for k in range(ROWS_PER_TILE // EG):
        pltpu.sync_copy(rows0, acc_s.at[pl.ds(r0 + k * EG, EG)])
    pltpu.sync_copy(src3_hbm.at[c, s, pl.ds(0, NG_MIN)],
                    srcv.at[pl.ds(0, NG_MIN)])
    pltpu.sync_copy(dst3_hbm.at[c, s, pl.ds(0, NG_MIN)],
                    dstv.at[pl.ds(0, NG_MIN)])
    plsc.subcore_barrier()
    rows = (rows0, rows1, rows2)
    gsems = (g0, g1, g2)
    ssems = (s0, s1, s2)

    def pipeline(n):
        gcp = [None] * NBUF
        scp = [None] * NBUF
        for t in range(min(2, n)):
            gcp[t % NBUF] = pltpu.async_copy(
                h_hbm.at[srcv.at[t]], rows[t % NBUF], gsems[t % NBUF])
        for j in range(n):
            b = j % NBUF
            gcp[b].wait()
            scp[b] = pltpu.async_copy(rows[b], acc_s.at[dstv.at[j]],
                                      ssems[b], add=True)
            nj = j + 2
            if nj < n:
                b2 = nj % NBUF
                if scp[b2] is not None:
                    scp[b2].wait()
                gcp[b2] = pltpu.async_copy(h_hbm.at[srcv.at[nj]], rows[b2],
                                           gsems[b2])
        for w in range(max(n - 3, 0), n):
            scp[w % NBUF].wait()

    pipeline(NG_MIN)

    if NG_EXT:
        @pl.when(c == BIG_CORE)
        def _():
            pltpu.sync_copy(src3_hbm.at[c, s, pl.ds(NG_MIN, NG_EXT)],
                            srcv.at[pl.ds(0, NG_EXT)])
            pltpu.sync_copy(dst3_hbm.at[c, s, pl.ds(NG_MIN, NG_EXT)],
                            dstv.at[pl.ds(0, NG_EXT)])
            pipeline(NG_EXT)

    plsc.subcore_barrier()
    for k in range(ROWS_PER_TILE // EG):
        pltpu.sync_copy(acc_s.at[pl.ds(r0 + k * EG, EG)], rows0)
        pltpu.sync_copy(rows0, out_hbm.at[c, pl.ds(r0 + k * EG, EG)])


def _make_spmm(d):
    params = None
    if d != 128:
        params = pltpu.CompilerParams(use_tc_tiling_on_sc=False)
    return pl.kernel(
        _spmm_body,
        out_type=jax.ShapeDtypeStruct((NC, NACC, d), jnp.float32),
        mesh=_mesh,
        scratch_types=(
            [pltpu.VMEM((IDXB, EG), jnp.int32)] * 2
            + [pltpu.VMEM((EG, d), jnp.float32)] * NBUF
            + [pltpu.SemaphoreType.DMA] * (2 * NBUF)
            + [pltpu.VMEM_SHARED((NACC, d), jnp.float32)]
        ),
        compiler_params=params,
        name=f"sc_spmm_{d}",
    )


_sc_spmm = {64: _make_spmm(64), 128: _make_spmm(128)}


def _dis_body(deg_ref, o_ref):
    d = jnp.sum(deg_ref[...], axis=0) + 1.0
    o_ref[...] = lax.rsqrt(d)


def _tc_dis(deg2):
    nr = NACC // 128
    return pl.pallas_call(
        _dis_body,
        out_shape=jax.ShapeDtypeStruct((nr, 128), jnp.float32),
    )(deg2.reshape(NC, nr, 128))


def _mm_body(x_ref, w_ref, dis_ref, o_ref):
    p = lax.dot_general(x_ref[...], w_ref[...], (((1,), (0,)), ((), ())),
                        preferred_element_type=jnp.float32,
                        precision=lax.Precision.HIGHEST)
    o_ref[...] = p * dis_ref[...]


def _tc_mm(x, w, dis, bm=1000):
    m, din = x.shape
    dout = w.shape[1]
    return pl.pallas_call(
        _mm_body,
        grid=(m // bm,),
        in_specs=[
            pl.BlockSpec((bm, din), lambda i: (i, 0)),
            pl.BlockSpec((din, dout), lambda i: (0, 0)),
            pl.BlockSpec((bm, 1), lambda i: (i, 0)),
        ],
        out_specs=pl.BlockSpec((bm, dout), lambda i: (i, 0)),
        out_shape=jax.ShapeDtypeStruct((m, dout), jnp.float32),
    )(x, w, dis)


def _comb_body(s0_ref, s1_ref, t_ref, dis_ref, b_ref, o_ref, *, act):
    v = (s0_ref[0] + s1_ref[0] + t_ref[...]) * dis_ref[...] + b_ref[...]
    if act:
        v = jnp.maximum(v, 0.0)
    o_ref[...] = v


def _tc_comb(sp, t, dis, b, act, bm=1000):
    m, d = t.shape
    return pl.pallas_call(
        functools.partial(_comb_body, act=act),
        grid=(m // bm,),
        in_specs=[
            pl.BlockSpec((1, bm, d), lambda i: (0, i, 0)),
            pl.BlockSpec((1, bm, d), lambda i: (1, i, 0)),
            pl.BlockSpec((bm, d), lambda i: (i, 0)),
            pl.BlockSpec((bm, 1), lambda i: (i, 0)),
            pl.BlockSpec((1, d), lambda i: (0, 0)),
        ],
        out_specs=pl.BlockSpec((bm, d), lambda i: (i, 0)),
        out_shape=jax.ShapeDtypeStruct((m, d), jnp.float32),
    )(sp, sp, t, dis, b)


def _adj_body(ai_ref, aj_ref, o_ref):
    p = lax.dot_general(ai_ref[...], aj_ref[...], (((1,), (1,)), ((), ())),
                        preferred_element_type=jnp.float32,
                        precision=lax.Precision.HIGHEST)
    o_ref[...] = jax.nn.sigmoid(p)


def _tc_adj(a, bm=400):
    m, d = a.shape
    return pl.pallas_call(
        _adj_body,
        grid=(m // bm,),
        in_specs=[
            pl.BlockSpec((bm, d), lambda i: (i, 0)),
            pl.BlockSpec((m, d), lambda i: (0, 0)),
        ],
        out_specs=pl.BlockSpec((bm, m), lambda i: (i, 0)),
        out_shape=jax.ShapeDtypeStruct((m, m), jnp.float32),
    )(a, a)


def kernel(x, edge_index, W1, b1, W2, b2, W3, b3, W4, b4, W5, b5):
    ei = edge_index.astype(jnp.int32)
    cap0 = NS * NG0 * EG
    cap1 = NS * NG1 * EG
    padl = cap0 + cap1 - E

    def split3(v, fill):
        vp = jnp.concatenate([v, fill])
        p0 = vp[:cap0].reshape(NS, NG0, EG)
        p1 = vp[cap0:].reshape(NS, NG1, EG)
        p0 = jnp.pad(p0, ((0, 0), (0, IDXR - NG0), (0, 0)))
        p1 = jnp.pad(p1, ((0, 0), (0, IDXR - NG1), (0, 0)))
        return jnp.stack([p0, p1])

    src3 = split3(ei[0], jnp.zeros((padl,), jnp.int32))
    dst3 = split3(ei[1], N + jnp.arange(padl, dtype=jnp.int32) % (NACC - N))
    znode = jnp.zeros((NACC,), jnp.float32)
    ones_eg = jnp.ones((EG,), jnp.float32)
    zrow = {64: jnp.zeros((EG, 64), jnp.float32),
            128: jnp.zeros((EG, 128), jnp.float32)}

    deg2 = _sc_deg(dst3, znode, ones_eg)
    dis = _tc_dis(deg2).reshape(NACC)[:N].reshape(N, 1)

    def conv(h, W, b, act):
        d = W.shape[1]
        t = _tc_mm(h, W, dis)
        sp = _sc_spmm[d](t, src3, dst3, zrow[d])
        return _tc_comb(sp, t, dis, b.reshape(1, d), act)

    z = conv(x, W1, b1, True)
    z = conv(z, W2, b2, True)
    h3 = conv(z, W3, b3, True)
    x_hat = conv(h3, W4, b4, False)
    a = conv(z, W5, b5, True)
    adj_hat = _tc_adj(a)
    return (x_hat, adj_hat)
